# Initial kernel scaffold; baseline (speedup 1.0000x reference)
import jax
import jax.numpy as jnp
from jax.experimental import pallas as pl


def kernel(x, pos_table):
    raise NotImplementedError("write your pallas kernel here")


# speedup vs baseline: 1.9569x; 1.9569x over previous
import jax
import jax.numpy as jnp
from jax.experimental import pallas as pl


def _body(x_ref, p_ref, o_ref):
    o_ref[...] = x_ref[...] + p_ref[...]


def kernel(x, pos_table):
    B, S, D = x.shape
    S_TILE = 512
    pos = pos_table[:S]
    return pl.pallas_call(
        _body,
        grid=(S // S_TILE,),
        in_specs=[
            pl.BlockSpec((B, S_TILE, D), lambda i: (0, i, 0)),
            pl.BlockSpec((S_TILE, D), lambda i: (i, 0)),
        ],
        out_specs=pl.BlockSpec((B, S_TILE, D), lambda i: (0, i, 0)),
        out_shape=jax.ShapeDtypeStruct(x.shape, x.dtype),
    )(x, pos)
